# baseline (speedup 1.0000x reference)
import functools

import jax
import jax.numpy as jnp
from jax import lax
from jax.experimental import pallas as pl
from jax.experimental.pallas import tpu as pltpu
from jax.experimental.pallas import tpu_sc as plsc

_NUM_CORES = 2
_NUM_SUBCORES = 16
_NUM_WORKERS = _NUM_CORES * _NUM_SUBCORES
_ROWS_PER_CHUNK = 8


def _gather_pairs(wpairs, flat_pair_ids, batch, seq):
    pair_dim = wpairs.shape[1]
    rows_per_worker = batch // _NUM_WORKERS
    chunk = _ROWS_PER_CHUNK * seq
    mesh = plsc.VectorSubcoreMesh(core_axis_name="c", subcore_axis_name="s")

    @functools.partial(
        pl.kernel,
        mesh=mesh,
        out_type=jax.ShapeDtypeStruct((batch, seq, pair_dim), wpairs.dtype),
        scratch_types=[
            pltpu.VMEM((chunk,), jnp.int32),
            pltpu.VMEM((chunk, pair_dim), wpairs.dtype),
            pltpu.SemaphoreType.DMA,
        ],
    )
    def gather_kernel(table_hbm, idx_hbm, out_hbm, idx_v, rows_v, sem):
        wid = lax.axis_index("s") * _NUM_CORES + lax.axis_index("c")
        row0 = wid * rows_per_worker

        @pl.loop(0, rows_per_worker, step=_ROWS_PER_CHUNK)
        def _(r):
            pltpu.sync_copy(idx_hbm.at[pl.ds((row0 + r) * seq, chunk)], idx_v)
            pltpu.async_copy(table_hbm.at[idx_v], rows_v, sem).wait()
            for j in range(_ROWS_PER_CHUNK):
                pltpu.sync_copy(
                    rows_v.at[pl.ds(j * seq, seq)],
                    out_hbm.at[row0 + r + j],
                )

    return gather_kernel(wpairs, flat_pair_ids)


def kernel(token_ids, weight):
    batch, seq = token_ids.shape
    num_rows, dim = weight.shape
    wpairs = weight.reshape(num_rows // 2, 2 * dim)
    flat_pair_ids = lax.shift_right_logical(token_ids, 1).reshape(batch * seq)
    pairs = _gather_pairs(wpairs, flat_pair_ids, batch, seq)
    odd = (token_ids & 1)[:, :, None].astype(jnp.bool_)
    return jnp.where(odd, pairs[:, :, dim:], pairs[:, :, :dim])


# speedup vs baseline: 1.5555x; 1.5555x over previous
import functools

import jax
import jax.numpy as jnp
from jax import lax
from jax.experimental import pallas as pl
from jax.experimental.pallas import tpu as pltpu
from jax.experimental.pallas import tpu_sc as plsc

_NUM_CORES = 2
_NUM_SUBCORES = 16
_NUM_WORKERS = _NUM_CORES * _NUM_SUBCORES
_ROWS_PER_CHUNK = 8
_BUILD_BLOCK = 2048


def _build_table(weight):
    num_rows, dim = weight.shape
    wt = weight.T

    def body(wt_ref, out_ref):
        out_ref[:, 0:dim] = wt_ref[...].T

    return pl.pallas_call(
        body,
        grid=(pl.cdiv(num_rows, _BUILD_BLOCK),),
        in_specs=[pl.BlockSpec((dim, _BUILD_BLOCK), lambda i: (0, i))],
        out_specs=pl.BlockSpec((_BUILD_BLOCK, 128), lambda i: (i, 0)),
        out_shape=jax.ShapeDtypeStruct((num_rows, 128), weight.dtype),
    )(wt)


def _gather_rows(wpad, flat_ids, batch, seq):
    pad_dim = wpad.shape[1]
    rows_per_worker = batch // _NUM_WORKERS
    chunk = _ROWS_PER_CHUNK * seq
    mesh = plsc.VectorSubcoreMesh(core_axis_name="c", subcore_axis_name="s")

    @functools.partial(
        pl.kernel,
        mesh=mesh,
        out_type=jax.ShapeDtypeStruct((batch, seq, pad_dim), wpad.dtype),
        scratch_types=[
            pltpu.VMEM((chunk,), jnp.int32),
            pltpu.VMEM((chunk, pad_dim), wpad.dtype),
            pltpu.SemaphoreType.DMA,
        ],
    )
    def gather_kernel(table_hbm, idx_hbm, out_hbm, idx_v, rows_v, sem):
        wid = lax.axis_index("s") * _NUM_CORES + lax.axis_index("c")
        row0 = wid * rows_per_worker

        @pl.loop(0, rows_per_worker, step=_ROWS_PER_CHUNK)
        def _(r):
            pltpu.sync_copy(idx_hbm.at[pl.ds((row0 + r) * seq, chunk)], idx_v)
            pltpu.async_copy(table_hbm.at[idx_v], rows_v, sem).wait()
            for j in range(_ROWS_PER_CHUNK):
                pltpu.sync_copy(
                    rows_v.at[pl.ds(j * seq, seq)],
                    out_hbm.at[row0 + r + j],
                )

    return gather_kernel(wpad, flat_ids)


def kernel(token_ids, weight):
    batch, seq = token_ids.shape
    dim = weight.shape[1]
    flat_ids = token_ids.reshape(batch * seq)
    wpad = _build_table(weight)
    out_pad = _gather_rows(wpad, flat_ids, batch, seq)
    return out_pad[:, :, :dim]


# speedup vs baseline: 1.9165x; 1.2321x over previous
import functools

import jax
import jax.numpy as jnp
from jax import lax
from jax.experimental import pallas as pl
from jax.experimental.pallas import tpu as pltpu
from jax.experimental.pallas import tpu_sc as plsc

_NUM_CORES = 2
_NUM_SUBCORES = 16
_NUM_WORKERS = _NUM_CORES * _NUM_SUBCORES
_ROWS_PER_CHUNK = 8
_BUILD_BLOCK = 8192


def _build_table(weight):
    num_rows, dim = weight.shape
    wt = weight.T

    def body(wt_ref, out_ref):
        out_ref[:, 0:dim] = wt_ref[...].T

    return pl.pallas_call(
        body,
        grid=(pl.cdiv(num_rows, _BUILD_BLOCK),),
        in_specs=[pl.BlockSpec((dim, _BUILD_BLOCK), lambda i: (0, i))],
        out_specs=pl.BlockSpec((_BUILD_BLOCK, 128), lambda i: (i, 0)),
        out_shape=jax.ShapeDtypeStruct((num_rows, 128), weight.dtype),
    )(wt)


def _gather_rows(wpad, flat_ids, batch, seq):
    pad_dim = wpad.shape[1]
    rows_per_worker = batch // _NUM_WORKERS
    chunk = _ROWS_PER_CHUNK * seq
    mesh = plsc.VectorSubcoreMesh(core_axis_name="c", subcore_axis_name="s")

    @functools.partial(
        pl.kernel,
        mesh=mesh,
        out_type=jax.ShapeDtypeStruct((batch, seq, pad_dim), wpad.dtype),
        scratch_types=[
            pltpu.VMEM((chunk,), jnp.int32),
            pltpu.VMEM((chunk, pad_dim), wpad.dtype),
            pltpu.SemaphoreType.DMA,
        ],
    )
    def gather_kernel(table_hbm, idx_hbm, out_hbm, idx_v, rows_v, sem):
        wid = lax.axis_index("s") * _NUM_CORES + lax.axis_index("c")
        row0 = wid * rows_per_worker

        @pl.loop(0, rows_per_worker, step=_ROWS_PER_CHUNK)
        def _(r):
            pltpu.sync_copy(idx_hbm.at[pl.ds((row0 + r) * seq, chunk)], idx_v)
            pltpu.async_copy(table_hbm.at[idx_v], rows_v, sem).wait()
            for j in range(_ROWS_PER_CHUNK):
                pltpu.sync_copy(
                    rows_v.at[pl.ds(j * seq, seq)],
                    out_hbm.at[row0 + r + j],
                )

    return gather_kernel(wpad, flat_ids)


def kernel(token_ids, weight):
    batch, seq = token_ids.shape
    dim = weight.shape[1]
    flat_ids = token_ids.reshape(batch * seq)
    wpad = _build_table(weight)
    out_pad = _gather_rows(wpad, flat_ids, batch, seq)
    return out_pad[:, :, :dim]


# speedup vs baseline: 1.9653x; 1.0254x over previous
import functools

import jax
import jax.numpy as jnp
from jax import lax
from jax.experimental import pallas as pl
from jax.experimental.pallas import tpu as pltpu
from jax.experimental.pallas import tpu_sc as plsc

_NUM_CORES = 2
_NUM_SUBCORES = 16
_NUM_WORKERS = _NUM_CORES * _NUM_SUBCORES
_ROWS_PER_CHUNK = 8
_BUILD_BLOCK = 16384


def _build_table(weight):
    num_rows, dim = weight.shape
    wt = weight.T

    def body(wt_ref, out_ref):
        out_ref[:, 0:dim] = wt_ref[...].T

    return pl.pallas_call(
        body,
        grid=(pl.cdiv(num_rows, _BUILD_BLOCK),),
        in_specs=[pl.BlockSpec((dim, _BUILD_BLOCK), lambda i: (0, i))],
        out_specs=pl.BlockSpec((_BUILD_BLOCK, 128), lambda i: (i, 0)),
        out_shape=jax.ShapeDtypeStruct((num_rows, 128), weight.dtype),
    )(wt)


def _gather_rows(wpad, flat_ids, batch, seq):
    pad_dim = wpad.shape[1]
    rows_per_worker = batch // _NUM_WORKERS
    chunk = _ROWS_PER_CHUNK * seq
    mesh = plsc.VectorSubcoreMesh(core_axis_name="c", subcore_axis_name="s")

    @functools.partial(
        pl.kernel,
        mesh=mesh,
        out_type=jax.ShapeDtypeStruct((batch, seq, pad_dim), wpad.dtype),
        scratch_types=[
            pltpu.VMEM((chunk,), jnp.int32),
            pltpu.VMEM((chunk, pad_dim), wpad.dtype),
            pltpu.SemaphoreType.DMA,
        ],
    )
    def gather_kernel(table_hbm, idx_hbm, out_hbm, idx_v, rows_v, sem):
        wid = lax.axis_index("s") * _NUM_CORES + lax.axis_index("c")
        row0 = wid * rows_per_worker

        @pl.loop(0, rows_per_worker, step=_ROWS_PER_CHUNK)
        def _(r):
            pltpu.sync_copy(idx_hbm.at[pl.ds((row0 + r) * seq, chunk)], idx_v)
            pltpu.async_copy(table_hbm.at[idx_v], rows_v, sem).wait()
            for j in range(_ROWS_PER_CHUNK):
                pltpu.sync_copy(
                    rows_v.at[pl.ds(j * seq, seq)],
                    out_hbm.at[row0 + r + j],
                )

    return gather_kernel(wpad, flat_ids)


def kernel(token_ids, weight):
    batch, seq = token_ids.shape
    dim = weight.shape[1]
    flat_ids = token_ids.reshape(batch * seq)
    wpad = _build_table(weight)
    out_pad = _gather_rows(wpad, flat_ids, batch, seq)
    return out_pad[:, :, :dim]


# speedup vs baseline: 1.9801x; 1.0075x over previous
import functools

import jax
import jax.numpy as jnp
from jax import lax
from jax.experimental import pallas as pl
from jax.experimental.pallas import tpu as pltpu
from jax.experimental.pallas import tpu_sc as plsc

_NUM_CORES = 2
_NUM_SUBCORES = 16
_NUM_WORKERS = _NUM_CORES * _NUM_SUBCORES
_ROWS_PER_CHUNK = 8
_BUILD_BLOCK = 32768


def _build_table(weight):
    num_rows, dim = weight.shape
    wt = weight.T

    def body(wt_ref, out_ref):
        out_ref[:, 0:dim] = wt_ref[...].T

    return pl.pallas_call(
        body,
        grid=(pl.cdiv(num_rows, _BUILD_BLOCK),),
        in_specs=[pl.BlockSpec((dim, _BUILD_BLOCK), lambda i: (0, i))],
        out_specs=pl.BlockSpec((_BUILD_BLOCK, 128), lambda i: (i, 0)),
        out_shape=jax.ShapeDtypeStruct((num_rows, 128), weight.dtype),
    )(wt)


def _gather_rows(wpad, flat_ids, batch, seq):
    pad_dim = wpad.shape[1]
    rows_per_worker = batch // _NUM_WORKERS
    chunk = _ROWS_PER_CHUNK * seq
    mesh = plsc.VectorSubcoreMesh(core_axis_name="c", subcore_axis_name="s")

    @functools.partial(
        pl.kernel,
        mesh=mesh,
        out_type=jax.ShapeDtypeStruct((batch, seq, pad_dim), wpad.dtype),
        scratch_types=[
            pltpu.VMEM((chunk,), jnp.int32),
            pltpu.VMEM((chunk, pad_dim), wpad.dtype),
            pltpu.SemaphoreType.DMA,
        ],
    )
    def gather_kernel(table_hbm, idx_hbm, out_hbm, idx_v, rows_v, sem):
        wid = lax.axis_index("s") * _NUM_CORES + lax.axis_index("c")
        row0 = wid * rows_per_worker

        @pl.loop(0, rows_per_worker, step=_ROWS_PER_CHUNK)
        def _(r):
            pltpu.sync_copy(idx_hbm.at[pl.ds((row0 + r) * seq, chunk)], idx_v)
            pltpu.async_copy(table_hbm.at[idx_v], rows_v, sem).wait()
            for j in range(_ROWS_PER_CHUNK):
                pltpu.sync_copy(
                    rows_v.at[pl.ds(j * seq, seq)],
                    out_hbm.at[row0 + r + j],
                )

    return gather_kernel(wpad, flat_ids)


def kernel(token_ids, weight):
    batch, seq = token_ids.shape
    dim = weight.shape[1]
    flat_ids = token_ids.reshape(batch * seq)
    wpad = _build_table(weight)
    out_pad = _gather_rows(wpad, flat_ids, batch, seq)
    return out_pad[:, :, :dim]
